# baseline (speedup 1.0000x reference)
import functools

import jax
import jax.numpy as jnp
from jax import lax
from jax.experimental import pallas as pl
from jax.experimental.pallas import tpu as pltpu
from jax.experimental.pallas import tpu_sc as plsc

B = 4096
L_SEQ = 200
D = 64
HALF = L_SEQ // 2

NC = 2
NS = 16
NW = NC * NS
RPW = B // NW


def _sc_pool_sums(emb_table, ids2):
    mesh = plsc.VectorSubcoreMesh(
        core_axis_name="c", subcore_axis_name="s", num_cores=NC, num_subcores=NS
    )

    @functools.partial(
        pl.kernel,
        mesh=mesh,
        compiler_params=pltpu.CompilerParams(use_tc_tiling_on_sc=False),
        out_type=jax.ShapeDtypeStruct((B, D), jnp.float32),
        scratch_types=[
            pltpu.VMEM((2 * RPW, HALF), jnp.int32),
            pltpu.VMEM((2, L_SEQ, D), jnp.float32),
            pltpu.VMEM((RPW, D), jnp.float32),
            pltpu.SemaphoreType.DMA,
            pltpu.SemaphoreType.DMA,
        ],
    )
    def k(emb_hbm, ids_hbm, out_hbm, ids_v, rows_v, out_v, sem0, sem1):
        wid = lax.axis_index("s") * NC + lax.axis_index("c")
        base2 = wid * (2 * RPW)
        pltpu.sync_copy(ids_hbm.at[pl.ds(base2, 2 * RPW)], ids_v)

        def descs(row, buf):
            sem = sem0 if buf == 0 else sem1
            d0 = pltpu.make_async_copy(
                emb_hbm.at[ids_v.at[2 * row]],
                rows_v.at[buf, pl.ds(0, HALF)],
                sem,
            )
            d1 = pltpu.make_async_copy(
                emb_hbm.at[ids_v.at[2 * row + 1]],
                rows_v.at[buf, pl.ds(HALF, HALF)],
                sem,
            )
            return d0, d1

        def start(row, buf):
            d0, d1 = descs(row, buf)
            d0.start()
            d1.start()

        def wait(row, buf):
            d0, d1 = descs(row, buf)
            d0.wait()
            d1.wait()

        def accum(row, buf):
            zero = jnp.zeros((16,), jnp.float32)

            def body(l, accs):
                return tuple(
                    accs[g] + rows_v[buf, l, pl.ds(g * 16, 16)] for g in range(4)
                )

            accs = lax.fori_loop(0, L_SEQ, body, (zero,) * 4)
            for g in range(4):
                out_v[row, pl.ds(g * 16, 16)] = accs[g]

        start(0, 0)
        start(1, 1)

        def pair(p, carry):
            i0 = 2 * p
            wait(i0, 0)
            accum(i0, 0)
            start(i0 + 2, 0)
            wait(i0 + 1, 1)
            accum(i0 + 1, 1)
            start(i0 + 3, 1)
            return carry

        lax.fori_loop(0, RPW // 2 - 1, pair, 0)
        wait(RPW - 2, 0)
        accum(RPW - 2, 0)
        wait(RPW - 1, 1)
        accum(RPW - 1, 1)
        pltpu.sync_copy(out_v, out_hbm.at[pl.ds(wid * RPW, RPW)])

    return k(emb_table, ids2)


BS_A = 128


def _video_body(vid_ref, ids_ref, wv_ref, vb_ref, v_ref, den_ref):
    csum = [None, None, None]
    for t in range(4):
        for c in range(3):
            blk = vid_ref[:, t, c, :, :]
            s = jnp.sum(jnp.sum(blk, axis=2), axis=1)
            csum[c] = s if csum[c] is None else csum[c] + s
    wv = wv_ref[...]
    out = jnp.zeros((BS_A, D), jnp.float32) + vb_ref[...]
    for c in range(3):
        out = out + csum[c][:, None] * (1.0 / 1024.0) * wv[c : c + 1, :]
    v_ref[...] = out
    cnt = jnp.sum((ids_ref[...] != 0).astype(jnp.float32), axis=1, keepdims=True)
    den_ref[...] = jnp.maximum(cnt, 1.0)


def _tc_video(video, ids, vid_wt, vid_b2):
    grid = (B // BS_A,)
    return pl.pallas_call(
        _video_body,
        grid=grid,
        in_specs=[
            pl.BlockSpec((BS_A, 4, 3, 16, 16), lambda i: (i, 0, 0, 0, 0)),
            pl.BlockSpec((BS_A, L_SEQ), lambda i: (i, 0)),
            pl.BlockSpec((3, D), lambda i: (0, 0)),
            pl.BlockSpec((1, D), lambda i: (0, 0)),
        ],
        out_specs=[
            pl.BlockSpec((BS_A, D), lambda i: (i, 0)),
            pl.BlockSpec((BS_A, 1), lambda i: (i, 0)),
        ],
        out_shape=[
            jax.ShapeDtypeStruct((B, D), jnp.float32),
            jax.ShapeDtypeStruct((B, 1), jnp.float32),
        ],
    )(video, ids, vid_wt, vid_b2)


BS_B = 1024


def _final_body(pooled_ref, v_ref, den_ref, wt_ref, tb_ref, out_ref):
    i = pl.program_id(0)
    x = pooled_ref[...] / den_ref[...]
    x = jnp.dot(x, wt_ref[...], preferred_element_type=jnp.float32) + tb_ref[...]
    v = v_ref[...]
    vn = v / jnp.maximum(jnp.sqrt(jnp.sum(v * v, axis=1, keepdims=True)), 1e-12)
    xn = x / jnp.maximum(jnp.sqrt(jnp.sum(x * x, axis=1, keepdims=True)), 1e-12)
    cos = jnp.sum(vn * xn, axis=1) / jnp.maximum(
        jnp.sqrt(jnp.sum(vn * vn, axis=1)) * jnp.sqrt(jnp.sum(xn * xn, axis=1)),
        1e-8,
    )
    part = jnp.sum(1.0 - cos) * (1.0 / B)

    @pl.when(i == 0)
    def _():
        out_ref[...] = jnp.zeros_like(out_ref)

    out_ref[...] += jnp.reshape(part, (1, 1))


def _tc_final(pooled, v, den, txt_wt, txt_b2):
    grid = (B // BS_B,)
    out = pl.pallas_call(
        _final_body,
        grid=grid,
        in_specs=[
            pl.BlockSpec((BS_B, D), lambda i: (i, 0)),
            pl.BlockSpec((BS_B, D), lambda i: (i, 0)),
            pl.BlockSpec((BS_B, 1), lambda i: (i, 0)),
            pl.BlockSpec((D, D), lambda i: (0, 0)),
            pl.BlockSpec((1, D), lambda i: (0, 0)),
        ],
        out_specs=pl.BlockSpec((1, 1), lambda i: (0, 0)),
        out_shape=jax.ShapeDtypeStruct((1, 1), jnp.float32),
    )(pooled, v, den, txt_wt, txt_b2)
    return out[0, 0]


def kernel(video, caption_ids, emb_table, txt_w, txt_b, vid_w, vid_b):
    ids = caption_ids.astype(jnp.int32)
    pooled = _sc_pool_sums(emb_table, ids.reshape(B * 2, HALF))
    v, den = _tc_video(video, ids, vid_w.T, vid_b.reshape(1, D))
    return _tc_final(pooled, v, den, txt_w.T, txt_b.reshape(1, D))


# speedup vs baseline: 1.3635x; 1.3635x over previous
import functools

import jax
import jax.numpy as jnp
from jax import lax
from jax.experimental import pallas as pl
from jax.experimental.pallas import tpu as pltpu
from jax.experimental.pallas import tpu_sc as plsc

B = 4096
L_SEQ = 200
D = 64
HALF = L_SEQ // 2

NC = 2
NS = 16
NW = NC * NS
RPW = B // NW


def _sc_pool_sums(emb_table, ids2):
    mesh = plsc.VectorSubcoreMesh(
        core_axis_name="c", subcore_axis_name="s", num_cores=NC, num_subcores=NS
    )

    @functools.partial(
        pl.kernel,
        mesh=mesh,
        compiler_params=pltpu.CompilerParams(use_tc_tiling_on_sc=False),
        out_type=jax.ShapeDtypeStruct((B, D), jnp.float32),
        scratch_types=[
            pltpu.VMEM((2 * RPW, HALF), jnp.int32),
            pltpu.VMEM((2, L_SEQ, D), jnp.float32),
            pltpu.VMEM((RPW, D), jnp.float32),
            pltpu.SemaphoreType.DMA,
            pltpu.SemaphoreType.DMA,
        ],
    )
    def k(emb_hbm, ids_hbm, out_hbm, ids_v, rows_v, out_v, sem0, sem1):
        wid = lax.axis_index("s") * NC + lax.axis_index("c")
        base2 = wid * (2 * RPW)
        pltpu.sync_copy(ids_hbm.at[pl.ds(base2, 2 * RPW)], ids_v)

        def descs(row, buf):
            sem = sem0 if buf == 0 else sem1
            d0 = pltpu.make_async_copy(
                emb_hbm.at[ids_v.at[2 * row]],
                rows_v.at[buf, pl.ds(0, HALF)],
                sem,
            )
            d1 = pltpu.make_async_copy(
                emb_hbm.at[ids_v.at[2 * row + 1]],
                rows_v.at[buf, pl.ds(HALF, HALF)],
                sem,
            )
            return d0, d1

        def start(row, buf):
            d0, d1 = descs(row, buf)
            d0.start()
            d1.start()

        def wait(row, buf):
            d0, d1 = descs(row, buf)
            d0.wait()
            d1.wait()

        def accum(row, buf):
            zero = jnp.zeros((16,), jnp.float32)

            def body(l, accs):
                return tuple(
                    accs[g] + rows_v[buf, l, pl.ds(g * 16, 16)] for g in range(4)
                )

            accs = lax.fori_loop(0, L_SEQ, body, (zero,) * 4)
            for g in range(4):
                out_v[row, pl.ds(g * 16, 16)] = accs[g]

        start(0, 0)
        start(1, 1)

        def pair(p, carry):
            i0 = 2 * p
            wait(i0, 0)
            accum(i0, 0)
            start(i0 + 2, 0)
            wait(i0 + 1, 1)
            accum(i0 + 1, 1)
            start(i0 + 3, 1)
            return carry

        lax.fori_loop(0, RPW // 2 - 1, pair, 0)
        wait(RPW - 2, 0)
        accum(RPW - 2, 0)
        wait(RPW - 1, 1)
        accum(RPW - 1, 1)
        pltpu.sync_copy(out_v, out_hbm.at[pl.ds(wid * RPW, RPW)])

    return k(emb_table, ids2)


BL = 512


def _video_body(vid_ref, ids_ref, wb_ref, vb_ref, v_ref, den_ref):
    v_ref[...] = (
        jnp.dot(wb_ref[...], vid_ref[...], preferred_element_type=jnp.float32)
        + vb_ref[...]
    )
    cnt = jnp.sum((ids_ref[...] != 0).astype(jnp.float32), axis=0, keepdims=True)
    den_ref[...] = jnp.maximum(cnt, 1.0)


def _tc_video(vid2, ids_t, w_big, vid_b2):
    grid = (B // BL,)
    return pl.pallas_call(
        _video_body,
        grid=grid,
        in_specs=[
            pl.BlockSpec((12 * 256, BL), lambda i: (0, i)),
            pl.BlockSpec((L_SEQ, BL), lambda i: (0, i)),
            pl.BlockSpec((D, 12 * 256), lambda i: (0, 0)),
            pl.BlockSpec((D, 1), lambda i: (0, 0)),
        ],
        out_specs=[
            pl.BlockSpec((D, BL), lambda i: (0, i)),
            pl.BlockSpec((1, BL), lambda i: (0, i)),
        ],
        out_shape=[
            jax.ShapeDtypeStruct((D, B), jnp.float32),
            jax.ShapeDtypeStruct((1, B), jnp.float32),
        ],
    )(vid2, ids_t, w_big, vid_b2)


def _final_body(pooled_ref, v_ref, den_ref, wt_ref, tb_ref, out_ref):
    x = pooled_ref[...] / den_ref[...]
    x = (
        jnp.dot(wt_ref[...], x, preferred_element_type=jnp.float32)
        + tb_ref[...]
    )
    v = v_ref[...]
    vn = v / jnp.maximum(
        jnp.sqrt(jnp.sum(v * v, axis=0, keepdims=True)), 1e-12
    )
    xn = x / jnp.maximum(
        jnp.sqrt(jnp.sum(x * x, axis=0, keepdims=True)), 1e-12
    )
    cos = jnp.sum(vn * xn, axis=0, keepdims=True) / jnp.maximum(
        jnp.sqrt(jnp.sum(vn * vn, axis=0, keepdims=True))
        * jnp.sqrt(jnp.sum(xn * xn, axis=0, keepdims=True)),
        1e-8,
    )
    loss = jnp.sum(1.0 - cos) * (1.0 / B)
    out_ref[...] = jnp.reshape(loss, (1, 1))


def _tc_final(pooled_t, v_t, den_t, txt_w, txt_b2):
    out = pl.pallas_call(
        _final_body,
        grid=(1,),
        in_specs=[
            pl.BlockSpec((D, B), lambda i: (0, 0)),
            pl.BlockSpec((D, B), lambda i: (0, 0)),
            pl.BlockSpec((1, B), lambda i: (0, 0)),
            pl.BlockSpec((D, D), lambda i: (0, 0)),
            pl.BlockSpec((D, 1), lambda i: (0, 0)),
        ],
        out_specs=pl.BlockSpec((1, 1), lambda i: (0, 0)),
        out_shape=jax.ShapeDtypeStruct((1, 1), jnp.float32),
    )(pooled_t, v_t, den_t, txt_w, txt_b2)
    return out[0, 0]


def kernel(video, caption_ids, emb_table, txt_w, txt_b, vid_w, vid_b):
    ids = caption_ids.astype(jnp.int32)
    pooled = _sc_pool_sums(emb_table, ids.reshape(B * 2, HALF))
    vid2 = video.transpose(1, 2, 3, 4, 0).reshape(12 * 256, B)
    ids_t = ids.T
    w_big = jnp.tile(jnp.repeat(vid_w * (1.0 / 1024.0), 256, axis=1), (1, 4))
    v_t, den_t = _tc_video(vid2, ids_t, w_big, vid_b.reshape(D, 1))
    return _tc_final(pooled.T, v_t, den_t, txt_w, txt_b.reshape(D, 1))
